# baseline (device time: 245756 ns/iter reference)
import jax
import jax.numpy as jnp
from jax import lax
from jax.experimental import pallas as pl
from jax.experimental.pallas import tpu as pltpu

N_DEV = 32


def _gelu(y):
    c = 0.7978845608028654
    return 0.5 * y * (1.0 + jnp.tanh(c * (y + 0.044715 * y * y * y)))


def kernel(x, w_mat):
    m_per, k = x.shape
    _, n_per = w_mat.shape

    def body(x_ref, w_ref, out_ref, comm_ref, send_sems, recv_sems):
        my_pos = lax.axis_index("i")
        left = (my_pos - 1) % N_DEV
        right = (my_pos + 1) % N_DEV

        barrier_sem = pltpu.get_barrier_semaphore()
        for nbr in [left, right]:
            pl.semaphore_signal(
                barrier_sem, inc=1,
                device_id=(nbr,), device_id_type=pl.DeviceIdType.MESH,
            )
        pl.semaphore_wait(barrier_sem, 2)

        out_ref[pl.ds(my_pos * m_per, m_per), :] = _gelu(
            jnp.dot(x_ref[:, :], w_ref[:, :], preferred_element_type=jnp.float32)
        )
        comm_ref[0, :, :] = x_ref[:, :]

        for h in range(N_DEV - 1):
            send_slot = h % 2
            recv_slot = (h + 1) % 2
            rdma = pltpu.make_async_remote_copy(
                src_ref=comm_ref.at[send_slot],
                dst_ref=comm_ref.at[recv_slot],
                send_sem=send_sems.at[send_slot],
                recv_sem=recv_sems.at[recv_slot],
                device_id=(right,),
                device_id_type=pl.DeviceIdType.MESH,
            )
            rdma.start()
            rdma.wait()

            origin = (my_pos - h - 1) % N_DEV
            out_ref[pl.ds(origin * m_per, m_per), :] = _gelu(
                jnp.dot(
                    comm_ref[recv_slot, :, :],
                    w_ref[:, :],
                    preferred_element_type=jnp.float32,
                )
            )

    return pl.pallas_call(
        body,
        out_shape=jax.ShapeDtypeStruct((N_DEV * m_per, n_per), jnp.float32),
        in_specs=[
            pl.BlockSpec(memory_space=pltpu.VMEM),
            pl.BlockSpec(memory_space=pltpu.VMEM),
        ],
        out_specs=pl.BlockSpec(memory_space=pltpu.VMEM),
        scratch_shapes=[
            pltpu.VMEM((2, m_per, k), jnp.float32),
            pltpu.SemaphoreType.DMA((2,)),
            pltpu.SemaphoreType.DMA((2,)),
        ],
        compiler_params=pltpu.CompilerParams(collective_id=0),
    )(x, w_mat)


# device time: 217242 ns/iter; 1.1313x vs baseline; 1.1313x over previous
import jax
import jax.numpy as jnp
from jax import lax
from jax.experimental import pallas as pl
from jax.experimental.pallas import tpu as pltpu

N_DEV = 32
CW_HOPS = N_DEV // 2
CCW_HOPS = N_DEV - 1 - CW_HOPS


def _gelu(y):
    c = 0.7978845608028654
    return 0.5 * y * (1.0 + jnp.tanh(c * (y + 0.044715 * y * y * y)))


def kernel(x, w_mat):
    m_per, k = x.shape
    _, n_per = w_mat.shape

    def body(
        x_ref, w_ref, out_ref,
        cw_ref, ccw_ref,
        cw_send, cw_recv, ccw_send, ccw_recv,
    ):
        my_pos = lax.axis_index("i")
        left = (my_pos - 1) % N_DEV
        right = (my_pos + 1) % N_DEV

        barrier_sem = pltpu.get_barrier_semaphore()
        for nbr in [left, right]:
            pl.semaphore_signal(
                barrier_sem, inc=1,
                device_id=(nbr,), device_id_type=pl.DeviceIdType.MESH,
            )
        pl.semaphore_wait(barrier_sem, 2)

        cw_ref[0, :, :] = x_ref[:, :]
        ccw_ref[0, :, :] = x_ref[:, :]
        out_ref[pl.ds(my_pos * m_per, m_per), :] = _gelu(
            jnp.dot(x_ref[:, :], w_ref[:, :], preferred_element_type=jnp.float32)
        )

        for h in range(CW_HOPS):
            s = h % 2
            r = (h + 1) % 2
            cw = pltpu.make_async_remote_copy(
                src_ref=cw_ref.at[s],
                dst_ref=cw_ref.at[r],
                send_sem=cw_send.at[s],
                recv_sem=cw_recv.at[r],
                device_id=(right,),
                device_id_type=pl.DeviceIdType.MESH,
            )
            cw.start()
            if h < CCW_HOPS:
                ccw = pltpu.make_async_remote_copy(
                    src_ref=ccw_ref.at[s],
                    dst_ref=ccw_ref.at[r],
                    send_sem=ccw_send.at[s],
                    recv_sem=ccw_recv.at[r],
                    device_id=(left,),
                    device_id_type=pl.DeviceIdType.MESH,
                )
                ccw.start()

            cw.wait()
            origin_cw = (my_pos - 1 - h) % N_DEV
            out_ref[pl.ds(origin_cw * m_per, m_per), :] = _gelu(
                jnp.dot(
                    cw_ref[r, :, :], w_ref[:, :],
                    preferred_element_type=jnp.float32,
                )
            )
            if h < CCW_HOPS:
                ccw.wait()
                origin_ccw = (my_pos + 1 + h) % N_DEV
                out_ref[pl.ds(origin_ccw * m_per, m_per), :] = _gelu(
                    jnp.dot(
                        ccw_ref[r, :, :], w_ref[:, :],
                        preferred_element_type=jnp.float32,
                    )
                )

    return pl.pallas_call(
        body,
        out_shape=jax.ShapeDtypeStruct((N_DEV * m_per, n_per), jnp.float32),
        in_specs=[
            pl.BlockSpec(memory_space=pltpu.VMEM),
            pl.BlockSpec(memory_space=pltpu.VMEM),
        ],
        out_specs=pl.BlockSpec(memory_space=pltpu.VMEM),
        scratch_shapes=[
            pltpu.VMEM((2, m_per, k), jnp.float32),
            pltpu.VMEM((2, m_per, k), jnp.float32),
            pltpu.SemaphoreType.DMA((2,)),
            pltpu.SemaphoreType.DMA((2,)),
            pltpu.SemaphoreType.DMA((2,)),
            pltpu.SemaphoreType.DMA((2,)),
        ],
        compiler_params=pltpu.CompilerParams(collective_id=0),
    )(x, w_mat)


# device time: 120640 ns/iter; 2.0371x vs baseline; 1.8007x over previous
import numpy as np

import jax
import jax.numpy as jnp
from jax import lax
from jax.experimental import pallas as pl
from jax.experimental.pallas import tpu as pltpu

N_DEV = 32
N_COL = 16
CW_HOPS = 8
CCW_HOPS = 7

_PLANE = [(0, 0), (1, 0), (1, 1), (0, 1), (0, 2), (1, 2), (1, 3), (0, 3)]
_COORDS = [(x, y, z) for z in range(4) for (x, y) in _PLANE]
_MESH_ID = {c: k for k, c in enumerate(_COORDS)}

_TOUR = [
    (0, 0), (0, 1), (0, 2), (0, 3),
    (1, 3), (1, 2), (1, 1),
    (2, 1), (2, 2), (2, 3),
    (3, 3), (3, 2), (3, 1), (3, 0),
    (2, 0), (1, 0),
]
_TOUR_POS = {c: p for p, c in enumerate(_TOUR)}


def _build_tables():
    right = np.zeros(N_DEV, np.int32)
    left = np.zeros(N_DEV, np.int32)
    base = np.zeros((N_DEV, N_COL), np.int32)
    for k, (x, y, z) in enumerate(_COORDS):
        p = _TOUR_POS[(y, z)]
        ny, nz = _TOUR[(p + 1) % N_COL]
        right[k] = _MESH_ID[(x, ny, nz)]
        py, pz = _TOUR[(p - 1) % N_COL]
        left[k] = _MESH_ID[(x, py, pz)]
        for d in range(N_COL):
            oy, oz = _TOUR[(p - d) % N_COL]
            base[k, d] = min(_MESH_ID[(0, oy, oz)], _MESH_ID[(1, oy, oz)])
    return right, left, base


_RIGHT_NP, _LEFT_NP, _BASE_NP = _build_tables()


def _gelu(y):
    c = 0.7978845608028654
    return 0.5 * y * (1.0 + jnp.tanh(c * (y + 0.044715 * y * y * y)))


def kernel(x, w_mat):
    m_per, k_dim = x.shape
    _, n_per = w_mat.shape
    sup = 2 * m_per

    def body(
        x_ref, w_ref, right_ref, left_ref, base_ref, out_ref,
        cw_ref, ccw_ref,
        xchg_send, xchg_recv,
        cw_send, cw_recv, ccw_send, ccw_recv,
    ):
        my_pos = lax.axis_index("i")
        partner = my_pos ^ 1
        right = right_ref[my_pos]
        left = left_ref[my_pos]

        barrier_sem = pltpu.get_barrier_semaphore()
        for nbr in [left, right, partner]:
            pl.semaphore_signal(
                barrier_sem, inc=1,
                device_id=(nbr,), device_id_type=pl.DeviceIdType.MESH,
            )
        pl.semaphore_wait(barrier_sem, 3)

        is_hi = (my_pos % 2) == 1

        @pl.when(jnp.logical_not(is_hi))
        def _():
            cw_ref[0, 0:m_per, :] = x_ref[:, :]
            xc = pltpu.make_async_remote_copy(
                src_ref=x_ref,
                dst_ref=cw_ref.at[0, pl.ds(0, m_per)],
                send_sem=xchg_send,
                recv_sem=xchg_recv,
                device_id=(partner,),
                device_id_type=pl.DeviceIdType.MESH,
            )
            xc.start()
            xc.wait()

        @pl.when(is_hi)
        def _():
            cw_ref[0, m_per:sup, :] = x_ref[:, :]
            xc = pltpu.make_async_remote_copy(
                src_ref=x_ref,
                dst_ref=cw_ref.at[0, pl.ds(m_per, m_per)],
                send_sem=xchg_send,
                recv_sem=xchg_recv,
                device_id=(partner,),
                device_id_type=pl.DeviceIdType.MESH,
            )
            xc.start()
            xc.wait()

        ccw_ref[0, :, :] = cw_ref[0, :, :]

        def col_out(base_id, chunk):
            out_ref[pl.ds(base_id * m_per, sup), :] = _gelu(
                jnp.dot(chunk, w_ref[:, :], preferred_element_type=jnp.float32)
            )

        def start_cw(h):
            s, r = h % 2, (h + 1) % 2
            rdma = pltpu.make_async_remote_copy(
                src_ref=cw_ref.at[s], dst_ref=cw_ref.at[r],
                send_sem=cw_send.at[s], recv_sem=cw_recv.at[r],
                device_id=(right,), device_id_type=pl.DeviceIdType.MESH,
            )
            rdma.start()
            return rdma

        def start_ccw(h):
            s, r = h % 2, (h + 1) % 2
            rdma = pltpu.make_async_remote_copy(
                src_ref=ccw_ref.at[s], dst_ref=ccw_ref.at[r],
                send_sem=ccw_send.at[s], recv_sem=ccw_recv.at[r],
                device_id=(left,), device_id_type=pl.DeviceIdType.MESH,
            )
            rdma.start()
            return rdma

        base_my = base_ref[my_pos, 0]

        cw_rdma = start_cw(0)
        ccw_rdma = start_ccw(0)
        col_out(base_my, cw_ref[0, :, :])
        cw_rdma.wait()
        ccw_rdma.wait()

        for h in range(1, CW_HOPS):
            cw_rdma = start_cw(h)
            if h < CCW_HOPS:
                ccw_rdma = start_ccw(h)
            r_prev = h % 2
            col_out(base_ref[my_pos, h], cw_ref[r_prev, :, :])
            col_out(base_ref[my_pos, N_COL - h], ccw_ref[r_prev, :, :])
            cw_rdma.wait()
            if h < CCW_HOPS:
                ccw_rdma.wait()

        r_last = CW_HOPS % 2
        col_out(base_ref[my_pos, CW_HOPS], cw_ref[r_last, :, :])
        col_out(base_ref[my_pos, N_COL - CCW_HOPS], ccw_ref[CCW_HOPS % 2, :, :])

    return pl.pallas_call(
        body,
        out_shape=jax.ShapeDtypeStruct((N_DEV * m_per, n_per), jnp.float32),
        in_specs=[
            pl.BlockSpec(memory_space=pltpu.VMEM),
            pl.BlockSpec(memory_space=pltpu.VMEM),
            pl.BlockSpec(memory_space=pltpu.SMEM),
            pl.BlockSpec(memory_space=pltpu.SMEM),
            pl.BlockSpec(memory_space=pltpu.SMEM),
        ],
        out_specs=pl.BlockSpec(memory_space=pltpu.VMEM),
        scratch_shapes=[
            pltpu.VMEM((2, sup, k_dim), jnp.float32),
            pltpu.VMEM((2, sup, k_dim), jnp.float32),
            pltpu.SemaphoreType.DMA,
            pltpu.SemaphoreType.DMA,
            pltpu.SemaphoreType.DMA((2,)),
            pltpu.SemaphoreType.DMA((2,)),
            pltpu.SemaphoreType.DMA((2,)),
            pltpu.SemaphoreType.DMA((2,)),
        ],
        compiler_params=pltpu.CompilerParams(collective_id=0),
    )(x, w_mat, jnp.asarray(_RIGHT_NP), jnp.asarray(_LEFT_NP), jnp.asarray(_BASE_NP))


# device time: 92460 ns/iter; 2.6580x vs baseline; 1.3048x over previous
import numpy as np

import jax
import jax.numpy as jnp
from jax import lax
from jax.experimental import pallas as pl
from jax.experimental.pallas import tpu as pltpu

N_DEV = 32
N_COL = 16
CW_HOPS = 8
CCW_HOPS = 7

_PLANE = [(0, 0), (1, 0), (1, 1), (0, 1), (0, 2), (1, 2), (1, 3), (0, 3)]
_COORDS = [(x, y, z) for z in range(4) for (x, y) in _PLANE]
_MESH_ID = {c: k for k, c in enumerate(_COORDS)}

_TOUR = [
    (0, 0), (0, 1), (0, 2), (0, 3),
    (1, 3), (1, 2), (1, 1),
    (2, 1), (2, 2), (2, 3),
    (3, 3), (3, 2), (3, 1), (3, 0),
    (2, 0), (1, 0),
]
_TOUR_POS = {c: p for p, c in enumerate(_TOUR)}


def _build_tables():
    right = np.zeros(N_DEV, np.int32)
    left = np.zeros(N_DEV, np.int32)
    layer = np.zeros(N_DEV, np.int32)
    base = np.zeros((N_DEV, N_COL), np.int32)
    for k, (x, y, z) in enumerate(_COORDS):
        layer[k] = x
        p = _TOUR_POS[(y, z)]
        ny, nz = _TOUR[(p + 1) % N_COL]
        right[k] = _MESH_ID[(x, ny, nz)]
        py, pz = _TOUR[(p - 1) % N_COL]
        left[k] = _MESH_ID[(x, py, pz)]
        for d in range(N_COL):
            oy, oz = _TOUR[(p - d) % N_COL]
            base[k, d] = min(_MESH_ID[(0, oy, oz)], _MESH_ID[(1, oy, oz)])
    return right, left, layer, base


_RIGHT_NP, _LEFT_NP, _LAYER_NP, _BASE_NP = _build_tables()


def _gelu(y):
    c = 0.7978845608028654
    return 0.5 * y * (1.0 + jnp.tanh(c * (y + 0.044715 * y * y * y)))


def kernel(x, w_mat):
    m_per, k_dim = x.shape
    _, n_per = w_mat.shape
    sup = 2 * m_per
    kh = k_dim // 2

    def body(
        x_ref, w_ref, right_ref, left_ref, layer_ref, base_ref, out_ref,
        cw_ref, ccw_ref, wcat_ref, pcat_ref, precv_ref,
        xchg_send, xchg_recv, wx_send, wx_recv, px_send, px_recv,
        cw_send, cw_recv, ccw_send, ccw_recv,
    ):
        my_pos = lax.axis_index("i")
        partner = my_pos ^ 1
        right = right_ref[my_pos]
        left = left_ref[my_pos]

        barrier_sem = pltpu.get_barrier_semaphore()
        for nbr in [left, right, partner]:
            pl.semaphore_signal(
                barrier_sem, inc=1,
                device_id=(nbr,), device_id_type=pl.DeviceIdType.MESH,
            )
        pl.semaphore_wait(barrier_sem, 3)

        row_off = (my_pos % 2) * m_per
        my_layer = layer_ref[my_pos]
        k_keep = my_layer * kh
        k_give = (1 - my_layer) * kh

        cw_ref[0, pl.ds(row_off, m_per), :] = x_ref[:, pl.ds(k_keep, kh)]
        wcat_ref[0, :, :] = w_ref[pl.ds(k_keep, kh), :]
        xc = pltpu.make_async_remote_copy(
            src_ref=x_ref.at[:, pl.ds(k_give, kh)],
            dst_ref=cw_ref.at[0, pl.ds(row_off, m_per)],
            send_sem=xchg_send, recv_sem=xchg_recv,
            device_id=(partner,), device_id_type=pl.DeviceIdType.MESH,
        )
        xc.start()
        wc = pltpu.make_async_remote_copy(
            src_ref=w_ref.at[pl.ds(k_give, kh)],
            dst_ref=wcat_ref.at[1],
            send_sem=wx_send, recv_sem=wx_recv,
            device_id=(partner,), device_id_type=pl.DeviceIdType.MESH,
        )
        wc.start()
        xc.wait()
        wc.wait()

        ccw_ref[0, :, :] = cw_ref[0, :, :]

        def col_partial(base_id, chunk):
            pcat_ref[0, pl.ds(base_id * m_per, sup), :] = jnp.dot(
                chunk, wcat_ref[0, :, :], preferred_element_type=jnp.float32
            )
            pcat_ref[1, pl.ds(base_id * m_per, sup), :] = jnp.dot(
                chunk, wcat_ref[1, :, :], preferred_element_type=jnp.float32
            )

        def start_cw(h):
            s, r = h % 2, (h + 1) % 2
            rdma = pltpu.make_async_remote_copy(
                src_ref=cw_ref.at[s], dst_ref=cw_ref.at[r],
                send_sem=cw_send.at[s], recv_sem=cw_recv.at[r],
                device_id=(right,), device_id_type=pl.DeviceIdType.MESH,
            )
            rdma.start()
            return rdma

        def start_ccw(h):
            s, r = h % 2, (h + 1) % 2
            rdma = pltpu.make_async_remote_copy(
                src_ref=ccw_ref.at[s], dst_ref=ccw_ref.at[r],
                send_sem=ccw_send.at[s], recv_sem=ccw_recv.at[r],
                device_id=(left,), device_id_type=pl.DeviceIdType.MESH,
            )
            rdma.start()
            return rdma

        cw_rdma = start_cw(0)
        ccw_rdma = start_ccw(0)
        col_partial(base_ref[my_pos, 0], cw_ref[0, :, :])
        cw_rdma.wait()
        ccw_rdma.wait()

        for h in range(1, CW_HOPS):
            cw_rdma = start_cw(h)
            if h < CCW_HOPS:
                ccw_rdma = start_ccw(h)
            r_prev = h % 2
            col_partial(base_ref[my_pos, h], cw_ref[r_prev, :, :])
            col_partial(base_ref[my_pos, N_COL - h], ccw_ref[r_prev, :, :])
            cw_rdma.wait()
            if h < CCW_HOPS:
                ccw_rdma.wait()

        r_last = CW_HOPS % 2
        col_partial(base_ref[my_pos, CW_HOPS], cw_ref[r_last, :, :])
        col_partial(base_ref[my_pos, N_COL - CCW_HOPS], ccw_ref[CCW_HOPS % 2, :, :])

        px = pltpu.make_async_remote_copy(
            src_ref=pcat_ref.at[1],
            dst_ref=precv_ref,
            send_sem=px_send, recv_sem=px_recv,
            device_id=(partner,), device_id_type=pl.DeviceIdType.MESH,
        )
        px.start()
        px.wait()
        out_ref[:, :] = _gelu(pcat_ref[0, :, :] + precv_ref[:, :])

    return pl.pallas_call(
        body,
        out_shape=jax.ShapeDtypeStruct((N_DEV * m_per, n_per), jnp.float32),
        in_specs=[
            pl.BlockSpec(memory_space=pltpu.VMEM),
            pl.BlockSpec(memory_space=pltpu.VMEM),
            pl.BlockSpec(memory_space=pltpu.SMEM),
            pl.BlockSpec(memory_space=pltpu.SMEM),
            pl.BlockSpec(memory_space=pltpu.SMEM),
            pl.BlockSpec(memory_space=pltpu.SMEM),
        ],
        out_specs=pl.BlockSpec(memory_space=pltpu.VMEM),
        scratch_shapes=[
            pltpu.VMEM((2, sup, kh), jnp.float32),
            pltpu.VMEM((2, sup, kh), jnp.float32),
            pltpu.VMEM((2, kh, n_per), jnp.float32),
            pltpu.VMEM((2, N_DEV * m_per, n_per), jnp.float32),
            pltpu.VMEM((N_DEV * m_per, n_per), jnp.float32),
            pltpu.SemaphoreType.DMA,
            pltpu.SemaphoreType.DMA,
            pltpu.SemaphoreType.DMA,
            pltpu.SemaphoreType.DMA,
            pltpu.SemaphoreType.DMA,
            pltpu.SemaphoreType.DMA,
            pltpu.SemaphoreType.DMA((2,)),
            pltpu.SemaphoreType.DMA((2,)),
            pltpu.SemaphoreType.DMA((2,)),
            pltpu.SemaphoreType.DMA((2,)),
        ],
        compiler_params=pltpu.CompilerParams(collective_id=0),
    )(x, w_mat, jnp.asarray(_RIGHT_NP), jnp.asarray(_LEFT_NP),
      jnp.asarray(_LAYER_NP), jnp.asarray(_BASE_NP))


# device time: 76344 ns/iter; 3.2191x vs baseline; 1.2111x over previous
import numpy as np

import jax
import jax.numpy as jnp
from jax import lax
from jax.experimental import pallas as pl
from jax.experimental.pallas import tpu as pltpu

N_DEV = 32
N_COL = 16
CW_HOPS = 8
CCW_HOPS = 7

_PLANE = [(0, 0), (1, 0), (1, 1), (0, 1), (0, 2), (1, 2), (1, 3), (0, 3)]
_COORDS = [(x, y, z) for z in range(4) for (x, y) in _PLANE]
_MESH_ID = {c: k for k, c in enumerate(_COORDS)}

_TOUR = [
    (0, 0), (0, 1), (0, 2), (0, 3),
    (1, 3), (1, 2), (1, 1),
    (2, 1), (2, 2), (2, 3),
    (3, 3), (3, 2), (3, 1), (3, 0),
    (2, 0), (1, 0),
]
_TOUR_POS = {c: p for p, c in enumerate(_TOUR)}


def _build_tables():
    right = np.zeros(N_DEV, np.int32)
    left = np.zeros(N_DEV, np.int32)
    layer = np.zeros(N_DEV, np.int32)
    base = np.zeros((N_DEV, N_COL), np.int32)
    for k, (x, y, z) in enumerate(_COORDS):
        layer[k] = x
        p = _TOUR_POS[(y, z)]
        ny, nz = _TOUR[(p + 1) % N_COL]
        right[k] = _MESH_ID[(x, ny, nz)]
        py, pz = _TOUR[(p - 1) % N_COL]
        left[k] = _MESH_ID[(x, py, pz)]
        for d in range(N_COL):
            oy, oz = _TOUR[(p - d) % N_COL]
            base[k, d] = min(_MESH_ID[(0, oy, oz)], _MESH_ID[(1, oy, oz)])
    return right, left, layer, base


_RIGHT_NP, _LEFT_NP, _LAYER_NP, _BASE_NP = _build_tables()


def _gelu(y):
    c = 0.7978845608028654
    return 0.5 * y * (1.0 + jnp.tanh(c * (y + 0.044715 * y * y * y)))


def kernel(x, w_mat):
    m_per, k_dim = x.shape
    _, n_per = w_mat.shape
    sup = 2 * m_per
    kh = k_dim // 2

    def body(
        x_ref, w_ref, right_ref, left_ref, layer_ref, base_ref, out_ref,
        cw_ref, ccw_ref, wcat_ref, pcat_ref, precv_ref,
        xchg_send, xchg_recv, wx_send, wx_recv, px_send, px_recv,
        cw_send, cw_recv, ccw_send, ccw_recv,
    ):
        my_pos = lax.axis_index("i")
        partner = my_pos ^ 1
        right = right_ref[my_pos]
        left = left_ref[my_pos]

        barrier_sem = pltpu.get_barrier_semaphore()
        for nbr in [left, right, partner]:
            pl.semaphore_signal(
                barrier_sem, inc=1,
                device_id=(nbr,), device_id_type=pl.DeviceIdType.MESH,
            )
        pl.semaphore_wait(barrier_sem, 3)

        row_off = (my_pos % 2) * m_per
        my_layer = layer_ref[my_pos]
        k_keep = my_layer * kh
        k_give = (1 - my_layer) * kh

        cw_ref[0, pl.ds(row_off, m_per), :] = x_ref[:, pl.ds(k_keep, kh)]
        wcat_ref[0, :, :] = w_ref[pl.ds(k_keep, kh), :]
        xc = pltpu.make_async_remote_copy(
            src_ref=x_ref.at[:, pl.ds(k_give, kh)],
            dst_ref=cw_ref.at[0, pl.ds(row_off, m_per)],
            send_sem=xchg_send, recv_sem=xchg_recv,
            device_id=(partner,), device_id_type=pl.DeviceIdType.MESH,
        )
        xc.start()
        wc = pltpu.make_async_remote_copy(
            src_ref=w_ref.at[pl.ds(k_give, kh)],
            dst_ref=wcat_ref.at[1],
            send_sem=wx_send, recv_sem=wx_recv,
            device_id=(partner,), device_id_type=pl.DeviceIdType.MESH,
        )
        wc.start()
        xc.wait()

        def start_cw(h):
            s, r = h % 2, (h + 1) % 2
            rdma = pltpu.make_async_remote_copy(
                src_ref=cw_ref.at[s], dst_ref=cw_ref.at[r],
                send_sem=cw_send.at[s], recv_sem=cw_recv.at[r],
                device_id=(right,), device_id_type=pl.DeviceIdType.MESH,
            )
            rdma.start()
            return rdma

        def start_ccw(h):
            src = cw_ref.at[0] if h == 0 else ccw_ref.at[h % 2]
            rdma = pltpu.make_async_remote_copy(
                src_ref=src, dst_ref=ccw_ref.at[(h + 1) % 2],
                send_sem=ccw_send.at[h % 2], recv_sem=ccw_recv.at[(h + 1) % 2],
                device_id=(left,), device_id_type=pl.DeviceIdType.MESH,
            )
            rdma.start()
            return rdma

        px_pending = []

        def col_partial(base_id, chunk):
            rows = pl.ds(base_id * m_per, sup)
            pcat_ref[0, rows, :] = jnp.dot(
                chunk, wcat_ref[0, :, :], preferred_element_type=jnp.float32
            )
            pcat_ref[1, rows, :] = jnp.dot(
                chunk, wcat_ref[1, :, :], preferred_element_type=jnp.float32
            )
            idx = len(px_pending)
            px = pltpu.make_async_remote_copy(
                src_ref=pcat_ref.at[1, rows],
                dst_ref=precv_ref.at[rows],
                send_sem=px_send.at[idx], recv_sem=px_recv.at[idx],
                device_id=(partner,), device_id_type=pl.DeviceIdType.MESH,
            )
            px.start()
            px_pending.append(px)

        cw_rdma = start_cw(0)
        ccw_rdma = start_ccw(0)
        wc.wait()
        col_partial(base_ref[my_pos, 0], cw_ref[0, :, :])
        cw_rdma.wait()
        ccw_rdma.wait()

        for h in range(1, CW_HOPS):
            cw_rdma = start_cw(h)
            if h < CCW_HOPS:
                ccw_rdma = start_ccw(h)
            r_prev = h % 2
            col_partial(base_ref[my_pos, h], cw_ref[r_prev, :, :])
            col_partial(base_ref[my_pos, N_COL - h], ccw_ref[r_prev, :, :])
            cw_rdma.wait()
            if h < CCW_HOPS:
                ccw_rdma.wait()

        col_partial(base_ref[my_pos, CW_HOPS], cw_ref[CW_HOPS % 2, :, :])

        for px in px_pending:
            px.wait()
        out_ref[:, :] = _gelu(pcat_ref[0, :, :] + precv_ref[:, :])

    return pl.pallas_call(
        body,
        out_shape=jax.ShapeDtypeStruct((N_DEV * m_per, n_per), jnp.float32),
        in_specs=[
            pl.BlockSpec(memory_space=pltpu.VMEM),
            pl.BlockSpec(memory_space=pltpu.VMEM),
            pl.BlockSpec(memory_space=pltpu.SMEM),
            pl.BlockSpec(memory_space=pltpu.SMEM),
            pl.BlockSpec(memory_space=pltpu.SMEM),
            pl.BlockSpec(memory_space=pltpu.SMEM),
        ],
        out_specs=pl.BlockSpec(memory_space=pltpu.VMEM),
        scratch_shapes=[
            pltpu.VMEM((2, sup, kh), jnp.float32),
            pltpu.VMEM((2, sup, kh), jnp.float32),
            pltpu.VMEM((2, kh, n_per), jnp.float32),
            pltpu.VMEM((2, N_DEV * m_per, n_per), jnp.float32),
            pltpu.VMEM((N_DEV * m_per, n_per), jnp.float32),
            pltpu.SemaphoreType.DMA,
            pltpu.SemaphoreType.DMA,
            pltpu.SemaphoreType.DMA,
            pltpu.SemaphoreType.DMA,
            pltpu.SemaphoreType.DMA((N_COL,)),
            pltpu.SemaphoreType.DMA((N_COL,)),
            pltpu.SemaphoreType.DMA((2,)),
            pltpu.SemaphoreType.DMA((2,)),
            pltpu.SemaphoreType.DMA((2,)),
            pltpu.SemaphoreType.DMA((2,)),
        ],
        compiler_params=pltpu.CompilerParams(collective_id=0),
    )(x, w_mat, jnp.asarray(_RIGHT_NP), jnp.asarray(_LEFT_NP),
      jnp.asarray(_LAYER_NP), jnp.asarray(_BASE_NP))


# device time: 61639 ns/iter; 3.9870x vs baseline; 1.2386x over previous
import numpy as np

import jax
import jax.numpy as jnp
from jax import lax
from jax.experimental import pallas as pl
from jax.experimental.pallas import tpu as pltpu

N_DEV = 32
N_COL = 16
CW_HOPS = 8
CCW_HOPS = 7

_PLANE = [(0, 0), (1, 0), (1, 1), (0, 1), (0, 2), (1, 2), (1, 3), (0, 3)]
_COORDS = [(x, y, z) for z in range(4) for (x, y) in _PLANE]
_MESH_ID = {c: k for k, c in enumerate(_COORDS)}

_TOUR = [
    (0, 0), (0, 1), (0, 2), (0, 3),
    (1, 3), (1, 2), (1, 1),
    (2, 1), (2, 2), (2, 3),
    (3, 3), (3, 2), (3, 1), (3, 0),
    (2, 0), (1, 0),
]
_TOUR_POS = {c: p for p, c in enumerate(_TOUR)}


def _build_tables():
    right = np.zeros(N_DEV, np.int32)
    left = np.zeros(N_DEV, np.int32)
    layer = np.zeros(N_DEV, np.int32)
    base = np.zeros((N_DEV, N_COL), np.int32)
    for k, (x, y, z) in enumerate(_COORDS):
        layer[k] = x
        p = _TOUR_POS[(y, z)]
        ny, nz = _TOUR[(p + 1) % N_COL]
        right[k] = _MESH_ID[(x, ny, nz)]
        py, pz = _TOUR[(p - 1) % N_COL]
        left[k] = _MESH_ID[(x, py, pz)]
        for d in range(N_COL):
            oy, oz = _TOUR[(p - d) % N_COL]
            base[k, d] = min(_MESH_ID[(0, oy, oz)], _MESH_ID[(1, oy, oz)])
    return right, left, layer, base


_RIGHT_NP, _LEFT_NP, _LAYER_NP, _BASE_NP = _build_tables()


def _gelu(y):
    c = 0.7978845608028654
    return 0.5 * y * (1.0 + jnp.tanh(c * (y + 0.044715 * y * y * y)))


def kernel(x, w_mat):
    m_per, k_dim = x.shape
    _, n_per = w_mat.shape
    sup = 2 * m_per
    kh = k_dim // 2

    def body(
        x_ref, w_ref, right_ref, left_ref, layer_ref, base_ref, out_ref,
        cw_ref, ccw_ref, wcat_ref, pcat_ref, precv_ref,
        xchg_send, xchg_recv, wx_send, wx_recv, px_send, px_recv,
        cw_send, cw_recv, ccw_send, ccw_recv,
    ):
        my_pos = lax.axis_index("i")
        partner = my_pos ^ 1
        right = right_ref[my_pos]
        left = left_ref[my_pos]

        barrier_sem = pltpu.get_barrier_semaphore()
        for nbr in [left, right, partner]:
            pl.semaphore_signal(
                barrier_sem, inc=1,
                device_id=(nbr,), device_id_type=pl.DeviceIdType.MESH,
            )
        pl.semaphore_wait(barrier_sem, 3)

        row_off = (my_pos % 2) * m_per
        my_layer = layer_ref[my_pos]
        k_keep = my_layer * kh
        k_give = (1 - my_layer) * kh

        cw_ref[0, pl.ds(row_off, m_per), :] = x_ref[:, pl.ds(k_keep, kh)]
        wcat_ref[0, :, :] = w_ref[pl.ds(k_keep, kh), :]
        xc = pltpu.make_async_remote_copy(
            src_ref=x_ref.at[:, pl.ds(k_give, kh)],
            dst_ref=cw_ref.at[0, pl.ds(row_off, m_per)],
            send_sem=xchg_send, recv_sem=xchg_recv,
            device_id=(partner,), device_id_type=pl.DeviceIdType.MESH,
        )
        xc.start()
        wc = pltpu.make_async_remote_copy(
            src_ref=w_ref.at[pl.ds(k_give, kh)],
            dst_ref=wcat_ref.at[1],
            send_sem=wx_send, recv_sem=wx_recv,
            device_id=(partner,), device_id_type=pl.DeviceIdType.MESH,
        )
        wc.start()
        xc.wait()

        def start_cw_sub(h, sub):
            s, r = h % 2, (h + 1) % 2
            rows = pl.ds(sub * m_per, m_per)
            rdma = pltpu.make_async_remote_copy(
                src_ref=cw_ref.at[s, rows], dst_ref=cw_ref.at[r, rows],
                send_sem=cw_send.at[s, sub], recv_sem=cw_recv.at[r, sub],
                device_id=(right,), device_id_type=pl.DeviceIdType.MESH,
            )
            rdma.start()
            return rdma

        def start_ccw_sub(h, sub):
            s, r = h % 2, (h + 1) % 2
            rows = pl.ds(sub * m_per, m_per)
            src = cw_ref.at[0, rows] if h == 0 else ccw_ref.at[s, rows]
            rdma = pltpu.make_async_remote_copy(
                src_ref=src, dst_ref=ccw_ref.at[r, rows],
                send_sem=ccw_send.at[s, sub], recv_sem=ccw_recv.at[r, sub],
                device_id=(left,), device_id_type=pl.DeviceIdType.MESH,
            )
            rdma.start()
            return rdma

        px_pending = []

        def px_stream(rows):
            idx = len(px_pending)
            px = pltpu.make_async_remote_copy(
                src_ref=pcat_ref.at[1, rows],
                dst_ref=precv_ref.at[rows],
                send_sem=px_send.at[idx], recv_sem=px_recv.at[idx],
                device_id=(partner,), device_id_type=pl.DeviceIdType.MESH,
            )
            px.start()
            px_pending.append(px)

        def col_partial(base_id, chunk):
            rows = pl.ds(base_id * m_per, sup)
            pcat_ref[0, rows, :] = jnp.dot(
                chunk, wcat_ref[0, :, :], preferred_element_type=jnp.float32
            )
            pcat_ref[1, rows, :] = jnp.dot(
                chunk, wcat_ref[1, :, :], preferred_element_type=jnp.float32
            )
            px_stream(rows)

        def antipode_partial(base_id):
            b = base_id * m_per
            top = cw_ref[0, 0:m_per, :]
            bot = ccw_ref[0, m_per:sup, :]
            for plane in (0, 1):
                pcat_ref[plane, pl.ds(b, m_per), :] = jnp.dot(
                    top, wcat_ref[plane, :, :],
                    preferred_element_type=jnp.float32,
                )
                pcat_ref[plane, pl.ds(b + m_per, m_per), :] = jnp.dot(
                    bot, wcat_ref[plane, :, :],
                    preferred_element_type=jnp.float32,
                )
            px_stream(pl.ds(b, sup))

        cw_cur = [start_cw_sub(0, 0), start_cw_sub(0, 1)]
        ccw_cur = [start_ccw_sub(0, 0), start_ccw_sub(0, 1)]
        wc.wait()
        col_partial(base_ref[my_pos, 0], cw_ref[0, :, :])

        for h in range(1, 7):
            cw_prev, ccw_prev = cw_cur, ccw_cur
            cw_prev[0].wait()
            a0 = start_cw_sub(h, 0)
            cw_prev[1].wait()
            a1 = start_cw_sub(h, 1)
            ccw_prev[0].wait()
            b0 = start_ccw_sub(h, 0)
            ccw_prev[1].wait()
            b1 = start_ccw_sub(h, 1)
            cw_cur, ccw_cur = [a0, a1], [b0, b1]
            r_prev = h % 2
            col_partial(base_ref[my_pos, h], cw_ref[r_prev, :, :])
            col_partial(base_ref[my_pos, N_COL - h], ccw_ref[r_prev, :, :])

        cw_cur[0].wait()
        cw7a = start_cw_sub(7, 0)
        cw_cur[1].wait()
        ccw_cur[1].wait()
        ccw7b = start_ccw_sub(7, 1)
        ccw_cur[0].wait()
        col_partial(base_ref[my_pos, 7], cw_ref[1, :, :])
        col_partial(base_ref[my_pos, 9], ccw_ref[1, :, :])

        cw7a.wait()
        ccw7b.wait()
        antipode_partial(base_ref[my_pos, 8])

        for px in px_pending:
            px.wait()
        out_ref[:, :] = _gelu(pcat_ref[0, :, :] + precv_ref[:, :])

    return pl.pallas_call(
        body,
        out_shape=jax.ShapeDtypeStruct((N_DEV * m_per, n_per), jnp.float32),
        in_specs=[
            pl.BlockSpec(memory_space=pltpu.VMEM),
            pl.BlockSpec(memory_space=pltpu.VMEM),
            pl.BlockSpec(memory_space=pltpu.SMEM),
            pl.BlockSpec(memory_space=pltpu.SMEM),
            pl.BlockSpec(memory_space=pltpu.SMEM),
            pl.BlockSpec(memory_space=pltpu.SMEM),
        ],
        out_specs=pl.BlockSpec(memory_space=pltpu.VMEM),
        scratch_shapes=[
            pltpu.VMEM((2, sup, kh), jnp.float32),
            pltpu.VMEM((2, sup, kh), jnp.float32),
            pltpu.VMEM((2, kh, n_per), jnp.float32),
            pltpu.VMEM((2, N_DEV * m_per, n_per), jnp.float32),
            pltpu.VMEM((N_DEV * m_per, n_per), jnp.float32),
            pltpu.SemaphoreType.DMA,
            pltpu.SemaphoreType.DMA,
            pltpu.SemaphoreType.DMA,
            pltpu.SemaphoreType.DMA,
            pltpu.SemaphoreType.DMA((N_COL,)),
            pltpu.SemaphoreType.DMA((N_COL,)),
            pltpu.SemaphoreType.DMA((2, 2)),
            pltpu.SemaphoreType.DMA((2, 2)),
            pltpu.SemaphoreType.DMA((2, 2)),
            pltpu.SemaphoreType.DMA((2, 2)),
        ],
        compiler_params=pltpu.CompilerParams(collective_id=0),
    )(x, w_mat, jnp.asarray(_RIGHT_NP), jnp.asarray(_LEFT_NP),
      jnp.asarray(_LAYER_NP), jnp.asarray(_BASE_NP))
